# initial kernel scaffold (unmeasured)
import jax
import jax.numpy as jnp
from jax import lax
from jax.experimental import pallas as pl
from jax.experimental.pallas import tpu as pltpu

B, S, D = 2, 512, 2048
DC = 256
DC_SH = 128
H, DH, DR = 16, 128, 32
BS = B * S
SCALE = (DH + DR) ** -0.5

_VMEM = pl.BlockSpec(memory_space=pltpu.VMEM)


def _gather_body(x_ref, wdkv_ref, wuk_ref, wuv_ref,
                 c_ref, wukf_ref, wuvf_ref,
                 send_sems, recv_sems):
    my_x = lax.axis_index("x")
    my_y = lax.axis_index("y")
    nbr = (my_x, 1 - my_y)

    barrier = pltpu.get_barrier_semaphore()
    pl.semaphore_signal(barrier, inc=1, device_id=nbr,
                        device_id_type=pl.DeviceIdType.MESH)
    pl.semaphore_wait(barrier, 1)

    c_loc = jnp.dot(x_ref[...], wdkv_ref[...],
                    preferred_element_type=jnp.float32)
    off = my_y * DC_SH
    c_ref[:, pl.ds(off, DC_SH)] = c_loc
    wukf_ref[pl.ds(off, DC_SH), :] = wuk_ref[...]
    wuvf_ref[pl.ds(off, DC_SH), :] = wuv_ref[...]

    copies = [
        pltpu.make_async_remote_copy(
            src_ref=c_ref.at[:, pl.ds(off, DC_SH)],
            dst_ref=c_ref.at[:, pl.ds(off, DC_SH)],
            send_sem=send_sems.at[0], recv_sem=recv_sems.at[0],
            device_id=nbr, device_id_type=pl.DeviceIdType.MESH),
        pltpu.make_async_remote_copy(
            src_ref=wukf_ref.at[pl.ds(off, DC_SH), :],
            dst_ref=wukf_ref.at[pl.ds(off, DC_SH), :],
            send_sem=send_sems.at[1], recv_sem=recv_sems.at[1],
            device_id=nbr, device_id_type=pl.DeviceIdType.MESH),
        pltpu.make_async_remote_copy(
            src_ref=wuvf_ref.at[pl.ds(off, DC_SH), :],
            dst_ref=wuvf_ref.at[pl.ds(off, DC_SH), :],
            send_sem=send_sems.at[2], recv_sem=recv_sems.at[2],
            device_id=nbr, device_id_type=pl.DeviceIdType.MESH),
    ]
    for cp in copies:
        cp.start()
    for cp in copies:
        cp.wait()


def _gather(x2d, wdkv, wuk, wuv):
    return pl.pallas_call(
        _gather_body,
        out_shape=(
            jax.ShapeDtypeStruct((BS, DC), jnp.float32),
            jax.ShapeDtypeStruct((DC, D), jnp.float32),
            jax.ShapeDtypeStruct((DC, D), jnp.float32),
        ),
        in_specs=[_VMEM] * 4,
        out_specs=(_VMEM,) * 3,
        scratch_shapes=[
            pltpu.SemaphoreType.DMA((3,)),
            pltpu.SemaphoreType.DMA((3,)),
        ],
        compiler_params=pltpu.CompilerParams(collective_id=0),
    )(x2d, wdkv, wuk, wuv)


def _matmul_body(a_ref, b_ref, o_ref):
    o_ref[...] = jnp.dot(a_ref[...], b_ref[...],
                         preferred_element_type=jnp.float32)


def _matmul(a, b, block_n):
    m, k = a.shape
    _, n = b.shape
    return pl.pallas_call(
        _matmul_body,
        grid=(n // block_n,),
        in_specs=[
            pl.BlockSpec((m, k), lambda j: (0, 0)),
            pl.BlockSpec((k, block_n), lambda j: (0, j)),
        ],
        out_specs=pl.BlockSpec((m, block_n), lambda j: (0, j)),
        out_shape=jax.ShapeDtypeStruct((m, n), jnp.float32),
    )(a, b)


def _kv_body(c_ref, wuk_ref, wuv_ref, k_ref, v_ref):
    c = c_ref[...]
    k_ref[...] = jnp.dot(c, wuk_ref[...], preferred_element_type=jnp.float32)
    v_ref[...] = jnp.dot(c, wuv_ref[...], preferred_element_type=jnp.float32)


def _kv(c, wuk, wuv, block_n=512):
    return pl.pallas_call(
        _kv_body,
        grid=(D // block_n,),
        in_specs=[
            pl.BlockSpec((BS, DC), lambda j: (0, 0)),
            pl.BlockSpec((DC, block_n), lambda j: (0, j)),
            pl.BlockSpec((DC, block_n), lambda j: (0, j)),
        ],
        out_specs=(
            pl.BlockSpec((BS, block_n), lambda j: (0, j)),
            pl.BlockSpec((BS, block_n), lambda j: (0, j)),
        ),
        out_shape=(
            jax.ShapeDtypeStruct((BS, D), jnp.float32),
            jax.ShapeDtypeStruct((BS, D), jnp.float32),
        ),
    )(c, wuk, wuv)


def _attn_body(q_ref, k_ref, v_ref, qr_ref, kr_ref, o_ref):
    q = q_ref[...]
    k = k_ref[...]
    qr = qr_ref[...]
    kr = kr_ref[...]
    dn = (((1,), (1,)), ((), ()))
    s = lax.dot_general(q, k, dn, preferred_element_type=jnp.float32)
    s = s + lax.dot_general(qr, kr, dn, preferred_element_type=jnp.float32)
    s = s * SCALE
    m = jnp.max(s, axis=-1, keepdims=True)
    p = jnp.exp(s - m)
    p = p / jnp.sum(p, axis=-1, keepdims=True)
    o_ref[...] = jnp.dot(p, v_ref[...], preferred_element_type=jnp.float32)


def _attention(q, k, v, qr, kr):
    return pl.pallas_call(
        _attn_body,
        grid=(B, H),
        in_specs=[
            pl.BlockSpec((S, DH), lambda b, h: (b, h)),
            pl.BlockSpec((S, DH), lambda b, h: (b, h)),
            pl.BlockSpec((S, DH), lambda b, h: (b, h)),
            pl.BlockSpec((S, DR), lambda b, h: (b, h)),
            pl.BlockSpec((S, DR), lambda b, h: (b, 0)),
        ],
        out_specs=pl.BlockSpec((S, DH), lambda b, h: (b, h)),
        out_shape=jax.ShapeDtypeStruct((BS, H * DH), jnp.float32),
    )(q, k, v, qr, kr)


def kernel(x, Wdkv, Wuk, Wuv, Wq, Wqr, Wkr, Wo):
    x2d = x.reshape(BS, D)

    c, wuk_f, wuv_f = _gather(x2d, Wdkv, Wuk, Wuv)
    k, v = _kv(c, wuk_f, wuv_f)
    q = _matmul(x2d, Wq, 512)
    qr = _matmul(x2d, Wqr, 512)
    kr = _matmul(x2d, Wkr, 32)
    o = _attention(q, k, v, qr, kr)
    out = _matmul(o, Wo, 512)
    return out.reshape(B, S, D)


# baseline (device time: 112934 ns/iter reference)
import jax
import jax.numpy as jnp
from jax import lax
from jax.experimental import pallas as pl
from jax.experimental.pallas import tpu as pltpu

B, S, D = 2, 512, 2048
DC = 256
DC_SH = 128
H, DH, DR = 16, 128, 32
BS = B * S
SCALE = (DH + DR) ** -0.5

_VMEM = pl.BlockSpec(memory_space=pltpu.VMEM)


def _gather_body(x_ref, wdkv_ref, wuk_ref, wuv_ref,
                 c_ref, wukf_ref, wuvf_ref,
                 send_sems, recv_sems):
    my_x = lax.axis_index("x")
    my_y = lax.axis_index("y")
    nbr = (my_x, 1 - my_y)

    barrier = pltpu.get_barrier_semaphore()
    pl.semaphore_signal(barrier, inc=1, device_id=nbr,
                        device_id_type=pl.DeviceIdType.MESH)
    pl.semaphore_wait(barrier, 1)

    c_loc = jnp.dot(x_ref[...], wdkv_ref[...],
                    preferred_element_type=jnp.float32)
    off = my_y * DC_SH
    c_ref[:, pl.ds(off, DC_SH)] = c_loc
    wukf_ref[pl.ds(off, DC_SH), :] = wuk_ref[...]
    wuvf_ref[pl.ds(off, DC_SH), :] = wuv_ref[...]

    copies = [
        pltpu.make_async_remote_copy(
            src_ref=c_ref.at[:, pl.ds(off, DC_SH)],
            dst_ref=c_ref.at[:, pl.ds(off, DC_SH)],
            send_sem=send_sems.at[0], recv_sem=recv_sems.at[0],
            device_id=nbr, device_id_type=pl.DeviceIdType.MESH),
        pltpu.make_async_remote_copy(
            src_ref=wukf_ref.at[pl.ds(off, DC_SH), :],
            dst_ref=wukf_ref.at[pl.ds(off, DC_SH), :],
            send_sem=send_sems.at[1], recv_sem=recv_sems.at[1],
            device_id=nbr, device_id_type=pl.DeviceIdType.MESH),
        pltpu.make_async_remote_copy(
            src_ref=wuvf_ref.at[pl.ds(off, DC_SH), :],
            dst_ref=wuvf_ref.at[pl.ds(off, DC_SH), :],
            send_sem=send_sems.at[2], recv_sem=recv_sems.at[2],
            device_id=nbr, device_id_type=pl.DeviceIdType.MESH),
    ]
    for cp in copies:
        cp.start()
    for cp in copies:
        cp.wait()


def _gather(x2d, wdkv, wuk, wuv):
    return pl.pallas_call(
        _gather_body,
        out_shape=(
            jax.ShapeDtypeStruct((BS, DC), jnp.float32),
            jax.ShapeDtypeStruct((DC, D), jnp.float32),
            jax.ShapeDtypeStruct((DC, D), jnp.float32),
        ),
        in_specs=[_VMEM] * 4,
        out_specs=(_VMEM,) * 3,
        scratch_shapes=[
            pltpu.SemaphoreType.DMA((3,)),
            pltpu.SemaphoreType.DMA((3,)),
        ],
        compiler_params=pltpu.CompilerParams(collective_id=0),
    )(x2d, wdkv, wuk, wuv)


def _matmul_body(a_ref, b_ref, o_ref):
    o_ref[...] = jnp.dot(a_ref[...], b_ref[...],
                         preferred_element_type=jnp.float32)


def _matmul(a, b, block_n):
    m, k = a.shape
    _, n = b.shape
    return pl.pallas_call(
        _matmul_body,
        grid=(n // block_n,),
        in_specs=[
            pl.BlockSpec((m, k), lambda j: (0, 0)),
            pl.BlockSpec((k, block_n), lambda j: (0, j)),
        ],
        out_specs=pl.BlockSpec((m, block_n), lambda j: (0, j)),
        out_shape=jax.ShapeDtypeStruct((m, n), jnp.float32),
    )(a, b)


def _kv_body(c_ref, wuk_ref, wuv_ref, k_ref, v_ref):
    c = c_ref[...]
    k_ref[...] = jnp.dot(c, wuk_ref[...], preferred_element_type=jnp.float32)
    v_ref[...] = jnp.dot(c, wuv_ref[...], preferred_element_type=jnp.float32)


def _kv(c, wuk, wuv, block_n=512):
    return pl.pallas_call(
        _kv_body,
        grid=(D // block_n,),
        in_specs=[
            pl.BlockSpec((BS, DC), lambda j: (0, 0)),
            pl.BlockSpec((DC, block_n), lambda j: (0, j)),
            pl.BlockSpec((DC, block_n), lambda j: (0, j)),
        ],
        out_specs=(
            pl.BlockSpec((BS, block_n), lambda j: (0, j)),
            pl.BlockSpec((BS, block_n), lambda j: (0, j)),
        ),
        out_shape=(
            jax.ShapeDtypeStruct((BS, D), jnp.float32),
            jax.ShapeDtypeStruct((BS, D), jnp.float32),
        ),
    )(c, wuk, wuv)


def _attn_body(q_ref, k_ref, v_ref, qr_ref, kr_ref, o_ref):
    kr = kr_ref[...]
    dn = (((1,), (1,)), ((), ()))
    for h in range(H):
        q = q_ref[:, h * DH:(h + 1) * DH]
        k = k_ref[:, h * DH:(h + 1) * DH]
        qr = qr_ref[:, h * DR:(h + 1) * DR]
        s = lax.dot_general(q, k, dn, preferred_element_type=jnp.float32)
        s = s + lax.dot_general(qr, kr, dn, preferred_element_type=jnp.float32)
        s = s * SCALE
        m = jnp.max(s, axis=-1, keepdims=True)
        p = jnp.exp(s - m)
        p = p / jnp.sum(p, axis=-1, keepdims=True)
        o_ref[:, h * DH:(h + 1) * DH] = jnp.dot(
            p, v_ref[:, h * DH:(h + 1) * DH],
            preferred_element_type=jnp.float32)


def _attention(q, k, v, qr, kr):
    return pl.pallas_call(
        _attn_body,
        grid=(B,),
        in_specs=[
            pl.BlockSpec((S, H * DH), lambda b: (b, 0)),
            pl.BlockSpec((S, H * DH), lambda b: (b, 0)),
            pl.BlockSpec((S, H * DH), lambda b: (b, 0)),
            pl.BlockSpec((S, H * DR), lambda b: (b, 0)),
            pl.BlockSpec((S, DR), lambda b: (b, 0)),
        ],
        out_specs=pl.BlockSpec((S, H * DH), lambda b: (b, 0)),
        out_shape=jax.ShapeDtypeStruct((BS, H * DH), jnp.float32),
    )(q, k, v, qr, kr)


def kernel(x, Wdkv, Wuk, Wuv, Wq, Wqr, Wkr, Wo):
    x2d = x.reshape(BS, D)

    c, wuk_f, wuv_f = _gather(x2d, Wdkv, Wuk, Wuv)
    k, v = _kv(c, wuk_f, wuv_f)
    q = _matmul(x2d, Wq, 512)
    qr = _matmul(x2d, Wqr, 512)
    kr = _matmul(x2d, Wkr, 32)
    o = _attention(q, k, v, qr, kr)
    out = _matmul(o, Wo, 512)
    return out.reshape(B, S, D)


# device time: 99508 ns/iter; 1.1349x vs baseline; 1.1349x over previous
import jax
import jax.numpy as jnp
from jax import lax
from jax.experimental import pallas as pl
from jax.experimental.pallas import tpu as pltpu

B, S, D = 2, 512, 2048
DC = 256
DC_SH = 128
H, DH, DR = 16, 128, 32
BS = B * S
SCALE = (DH + DR) ** -0.5

_VMEM = pl.BlockSpec(memory_space=pltpu.VMEM)


def _gather_q_body(x_ref, wdkv_ref, wuk_ref, wuv_ref, wq_ref, wqr_ref,
                   wkr_ref,
                   c_ref, wukf_ref, wuvf_ref, q_ref, qr_ref, kr_ref,
                   send_sems, recv_sems):
    my_x = lax.axis_index("x")
    my_y = lax.axis_index("y")
    nbr = (my_x, 1 - my_y)

    barrier = pltpu.get_barrier_semaphore()
    pl.semaphore_signal(barrier, inc=1, device_id=nbr,
                        device_id_type=pl.DeviceIdType.MESH)
    pl.semaphore_wait(barrier, 1)

    x = x_ref[...]
    c_loc = jnp.dot(x, wdkv_ref[...], preferred_element_type=jnp.float32)
    off = my_y * DC_SH
    c_ref[:, pl.ds(off, DC_SH)] = c_loc
    wukf_ref[pl.ds(off, DC_SH), :] = wuk_ref[...]
    wuvf_ref[pl.ds(off, DC_SH), :] = wuv_ref[...]

    copies = [
        pltpu.make_async_remote_copy(
            src_ref=c_ref.at[:, pl.ds(off, DC_SH)],
            dst_ref=c_ref.at[:, pl.ds(off, DC_SH)],
            send_sem=send_sems.at[0], recv_sem=recv_sems.at[0],
            device_id=nbr, device_id_type=pl.DeviceIdType.MESH),
        pltpu.make_async_remote_copy(
            src_ref=wukf_ref.at[pl.ds(off, DC_SH), :],
            dst_ref=wukf_ref.at[pl.ds(off, DC_SH), :],
            send_sem=send_sems.at[1], recv_sem=recv_sems.at[1],
            device_id=nbr, device_id_type=pl.DeviceIdType.MESH),
        pltpu.make_async_remote_copy(
            src_ref=wuvf_ref.at[pl.ds(off, DC_SH), :],
            dst_ref=wuvf_ref.at[pl.ds(off, DC_SH), :],
            send_sem=send_sems.at[2], recv_sem=recv_sems.at[2],
            device_id=nbr, device_id_type=pl.DeviceIdType.MESH),
    ]
    for cp in copies:
        cp.start()

    q_ref[...] = jnp.dot(x, wq_ref[...], preferred_element_type=jnp.float32)
    qr_ref[...] = jnp.dot(x, wqr_ref[...], preferred_element_type=jnp.float32)
    kr_ref[...] = jnp.dot(x, wkr_ref[...], preferred_element_type=jnp.float32)

    for cp in copies:
        cp.wait()


def _gather_q(x2d, wdkv, wuk, wuv, wq, wqr, wkr):
    return pl.pallas_call(
        _gather_q_body,
        out_shape=(
            jax.ShapeDtypeStruct((BS, DC), jnp.float32),
            jax.ShapeDtypeStruct((DC, D), jnp.float32),
            jax.ShapeDtypeStruct((DC, D), jnp.float32),
            jax.ShapeDtypeStruct((BS, D), jnp.float32),
            jax.ShapeDtypeStruct((BS, H * DR), jnp.float32),
            jax.ShapeDtypeStruct((BS, DR), jnp.float32),
        ),
        in_specs=[_VMEM] * 7,
        out_specs=(_VMEM,) * 6,
        scratch_shapes=[
            pltpu.SemaphoreType.DMA((3,)),
            pltpu.SemaphoreType.DMA((3,)),
        ],
        compiler_params=pltpu.CompilerParams(
            collective_id=0, vmem_limit_bytes=64 * 1024 * 1024),
    )(x2d, wdkv, wuk, wuv, wq, wqr, wkr)


def _attn_body(c_ref, wuk_ref, wuv_ref, q_ref, qr_ref, kr_ref, o_ref):
    c = c_ref[...]
    kk = jnp.dot(c, wuk_ref[...], preferred_element_type=jnp.float32)
    vv = jnp.dot(c, wuv_ref[...], preferred_element_type=jnp.float32)
    kr = kr_ref[...]
    dn = (((1,), (1,)), ((), ()))
    for h in range(H):
        q = q_ref[:, h * DH:(h + 1) * DH]
        k = kk[:, h * DH:(h + 1) * DH]
        qr = qr_ref[:, h * DR:(h + 1) * DR]
        s = lax.dot_general(q, k, dn, preferred_element_type=jnp.float32)
        s = s + lax.dot_general(qr, kr, dn, preferred_element_type=jnp.float32)
        s = s * SCALE
        m = jnp.max(s, axis=-1, keepdims=True)
        p = jnp.exp(s - m)
        p = p / jnp.sum(p, axis=-1, keepdims=True)
        o_ref[:, h * DH:(h + 1) * DH] = jnp.dot(
            p, vv[:, h * DH:(h + 1) * DH],
            preferred_element_type=jnp.float32)


def _attention(c, wuk_f, wuv_f, q, qr, kr):
    return pl.pallas_call(
        _attn_body,
        grid=(B,),
        in_specs=[
            pl.BlockSpec((S, DC), lambda b: (b, 0)),
            pl.BlockSpec((DC, D), lambda b: (0, 0)),
            pl.BlockSpec((DC, D), lambda b: (0, 0)),
            pl.BlockSpec((S, H * DH), lambda b: (b, 0)),
            pl.BlockSpec((S, H * DR), lambda b: (b, 0)),
            pl.BlockSpec((S, DR), lambda b: (b, 0)),
        ],
        out_specs=pl.BlockSpec((S, H * DH), lambda b: (b, 0)),
        out_shape=jax.ShapeDtypeStruct((BS, H * DH), jnp.float32),
        compiler_params=pltpu.CompilerParams(
            vmem_limit_bytes=64 * 1024 * 1024),
    )(c, wuk_f, wuv_f, q, qr, kr)


def _matmul_body(a_ref, b_ref, o_ref):
    o_ref[...] = jnp.dot(a_ref[...], b_ref[...],
                         preferred_element_type=jnp.float32)


def _matmul(a, b, block_n):
    m, k = a.shape
    _, n = b.shape
    return pl.pallas_call(
        _matmul_body,
        grid=(n // block_n,),
        in_specs=[
            pl.BlockSpec((m, k), lambda j: (0, 0)),
            pl.BlockSpec((k, block_n), lambda j: (0, j)),
        ],
        out_specs=pl.BlockSpec((m, block_n), lambda j: (0, j)),
        out_shape=jax.ShapeDtypeStruct((m, n), jnp.float32),
        compiler_params=pltpu.CompilerParams(
            vmem_limit_bytes=64 * 1024 * 1024),
    )(a, b)


def kernel(x, Wdkv, Wuk, Wuv, Wq, Wqr, Wkr, Wo):
    x2d = x.reshape(BS, D)

    c, wuk_f, wuv_f, q, qr, kr = _gather_q(x2d, Wdkv, Wuk, Wuv, Wq, Wqr, Wkr)
    o = _attention(c, wuk_f, wuv_f, q, qr, kr)
    out = _matmul(o, Wo, 512)
    return out.reshape(B, S, D)


# device time: 78907 ns/iter; 1.4312x vs baseline; 1.2611x over previous
import jax
import jax.numpy as jnp
from jax import lax
from jax.experimental import pallas as pl
from jax.experimental.pallas import tpu as pltpu

B, S, D = 2, 512, 2048
DC = 256
DC_SH = 128
H, DH, DR = 16, 128, 32
BS = B * S
SCALE = (DH + DR) ** -0.5

_VMEM = pl.BlockSpec(memory_space=pltpu.VMEM)
_VMEM_LIMIT = 100 * 1024 * 1024


def _gather_q_body(x_ref, wdkv_ref, wuk_ref, wuv_ref, wq_ref, wqr_ref,
                   wkr_ref,
                   c_ref, wukf_ref, wuvf_ref, q_ref, qr_ref, kr_ref,
                   send_sems, recv_sems):
    my_x = lax.axis_index("x")
    my_y = lax.axis_index("y")
    nbr = (my_x, 1 - my_y)

    barrier = pltpu.get_barrier_semaphore()
    pl.semaphore_signal(barrier, inc=1, device_id=nbr,
                        device_id_type=pl.DeviceIdType.MESH)
    pl.semaphore_wait(barrier, 1)

    x = x_ref[...]
    c_loc = jnp.dot(x, wdkv_ref[...], preferred_element_type=jnp.float32)
    off = my_y * DC_SH
    c_ref[:, pl.ds(off, DC_SH)] = c_loc.astype(jnp.bfloat16)
    wukf_ref[pl.ds(off, DC_SH), :] = wuk_ref[...].astype(jnp.bfloat16)
    wuvf_ref[pl.ds(off, DC_SH), :] = wuv_ref[...].astype(jnp.bfloat16)

    copies = [
        pltpu.make_async_remote_copy(
            src_ref=c_ref.at[:, pl.ds(off, DC_SH)],
            dst_ref=c_ref.at[:, pl.ds(off, DC_SH)],
            send_sem=send_sems.at[0], recv_sem=recv_sems.at[0],
            device_id=nbr, device_id_type=pl.DeviceIdType.MESH),
        pltpu.make_async_remote_copy(
            src_ref=wukf_ref.at[pl.ds(off, DC_SH), :],
            dst_ref=wukf_ref.at[pl.ds(off, DC_SH), :],
            send_sem=send_sems.at[1], recv_sem=recv_sems.at[1],
            device_id=nbr, device_id_type=pl.DeviceIdType.MESH),
        pltpu.make_async_remote_copy(
            src_ref=wuvf_ref.at[pl.ds(off, DC_SH), :],
            dst_ref=wuvf_ref.at[pl.ds(off, DC_SH), :],
            send_sem=send_sems.at[2], recv_sem=recv_sems.at[2],
            device_id=nbr, device_id_type=pl.DeviceIdType.MESH),
    ]
    for cp in copies:
        cp.start()

    q_ref[...] = jnp.dot(
        x, wq_ref[...], preferred_element_type=jnp.float32
    ).astype(jnp.bfloat16)
    qr_ref[...] = jnp.dot(
        x, wqr_ref[...], preferred_element_type=jnp.float32
    ).astype(jnp.bfloat16)
    kr_ref[...] = jnp.dot(
        x, wkr_ref[...], preferred_element_type=jnp.float32
    ).astype(jnp.bfloat16)

    for cp in copies:
        cp.wait()


def _gather_q(x2d, wdkv, wuk, wuv, wq, wqr, wkr):
    return pl.pallas_call(
        _gather_q_body,
        out_shape=(
            jax.ShapeDtypeStruct((BS, DC), jnp.bfloat16),
            jax.ShapeDtypeStruct((DC, D), jnp.bfloat16),
            jax.ShapeDtypeStruct((DC, D), jnp.bfloat16),
            jax.ShapeDtypeStruct((BS, D), jnp.bfloat16),
            jax.ShapeDtypeStruct((BS, H * DR), jnp.bfloat16),
            jax.ShapeDtypeStruct((BS, DR), jnp.bfloat16),
        ),
        in_specs=[_VMEM] * 7,
        out_specs=(_VMEM,) * 6,
        scratch_shapes=[
            pltpu.SemaphoreType.DMA((3,)),
            pltpu.SemaphoreType.DMA((3,)),
        ],
        compiler_params=pltpu.CompilerParams(
            collective_id=0, vmem_limit_bytes=_VMEM_LIMIT),
    )(x2d, wdkv, wuk, wuv, wq, wqr, wkr)


def _attn_body(c_ref, wuk_ref, wuv_ref, q_ref, qr_ref, kr_ref, wo_ref,
               out_ref, o_scratch):
    c = c_ref[...]
    kk = jnp.dot(c, wuk_ref[...],
                 preferred_element_type=jnp.float32).astype(jnp.bfloat16)
    vv = jnp.dot(c, wuv_ref[...],
                 preferred_element_type=jnp.float32).astype(jnp.bfloat16)
    kr = kr_ref[...]
    dn = (((1,), (1,)), ((), ()))
    for h in range(H):
        q = q_ref[:, h * DH:(h + 1) * DH]
        k = kk[:, h * DH:(h + 1) * DH]
        qr = qr_ref[:, h * DR:(h + 1) * DR]
        s = lax.dot_general(q, k, dn, preferred_element_type=jnp.float32)
        s = s + lax.dot_general(qr, kr, dn, preferred_element_type=jnp.float32)
        s = s * SCALE
        m = jnp.max(s, axis=-1, keepdims=True)
        p = jnp.exp(s - m)
        p = (p / jnp.sum(p, axis=-1, keepdims=True)).astype(jnp.bfloat16)
        o_scratch[:, h * DH:(h + 1) * DH] = jnp.dot(
            p, vv[:, h * DH:(h + 1) * DH],
            preferred_element_type=jnp.float32).astype(jnp.bfloat16)
    out_ref[...] = jnp.dot(o_scratch[...], wo_ref[...],
                           preferred_element_type=jnp.float32)


def _attention(c, wuk_f, wuv_f, q, qr, kr, wo):
    return pl.pallas_call(
        _attn_body,
        grid=(B,),
        in_specs=[
            pl.BlockSpec((S, DC), lambda b: (b, 0)),
            pl.BlockSpec((DC, D), lambda b: (0, 0)),
            pl.BlockSpec((DC, D), lambda b: (0, 0)),
            pl.BlockSpec((S, H * DH), lambda b: (b, 0)),
            pl.BlockSpec((S, H * DR), lambda b: (b, 0)),
            pl.BlockSpec((S, DR), lambda b: (b, 0)),
            pl.BlockSpec((D, D), lambda b: (0, 0)),
        ],
        out_specs=pl.BlockSpec((S, D), lambda b: (b, 0)),
        out_shape=jax.ShapeDtypeStruct((BS, D), jnp.float32),
        scratch_shapes=[pltpu.VMEM((S, H * DH), jnp.bfloat16)],
        compiler_params=pltpu.CompilerParams(vmem_limit_bytes=_VMEM_LIMIT),
    )(c, wuk_f, wuv_f, q, qr, kr, wo)


def kernel(x, Wdkv, Wuk, Wuv, Wq, Wqr, Wkr, Wo):
    x2d = x.reshape(BS, D)

    c, wuk_f, wuv_f, q, qr, kr = _gather_q(x2d, Wdkv, Wuk, Wuv, Wq, Wqr, Wkr)
    out = _attention(c, wuk_f, wuv_f, q, qr, kr, Wo)
    return out.reshape(B, S, D)


# device time: 70462 ns/iter; 1.6028x vs baseline; 1.1199x over previous
import jax
import jax.numpy as jnp
from jax import lax
from jax.experimental import pallas as pl
from jax.experimental.pallas import tpu as pltpu

B, S, D = 2, 512, 2048
DC = 256
DC_SH = 128
H, DH, DR = 16, 128, 32
BS = B * S
SCALE = (DH + DR) ** -0.5
NT = 4
TD = D // NT

_VMEM = pl.BlockSpec(memory_space=pltpu.VMEM)
_VMEM_LIMIT = 100 * 1024 * 1024


def _gather_q_body(x_ref, wdkv_ref, wuk_ref, wuv_ref, wq_ref, wqr_ref,
                   wkr_ref, wo_ref,
                   c_ref, wukf_ref, wuvf_ref, q_ref, qr_ref, kr_ref,
                   wob_ref,
                   send_sems, recv_sems):
    my_x = lax.axis_index("x")
    my_y = lax.axis_index("y")
    nbr = (my_x, 1 - my_y)
    off = my_y * DC_SH
    j = pl.program_id(0)

    def _copies():
        return [
            pltpu.make_async_remote_copy(
                src_ref=c_ref.at[:, pl.ds(off, DC_SH)],
                dst_ref=c_ref.at[:, pl.ds(off, DC_SH)],
                send_sem=send_sems.at[0], recv_sem=recv_sems.at[0],
                device_id=nbr, device_id_type=pl.DeviceIdType.MESH),
            pltpu.make_async_remote_copy(
                src_ref=wukf_ref.at[pl.ds(off, DC_SH), :],
                dst_ref=wukf_ref.at[pl.ds(off, DC_SH), :],
                send_sem=send_sems.at[1], recv_sem=recv_sems.at[1],
                device_id=nbr, device_id_type=pl.DeviceIdType.MESH),
            pltpu.make_async_remote_copy(
                src_ref=wuvf_ref.at[pl.ds(off, DC_SH), :],
                dst_ref=wuvf_ref.at[pl.ds(off, DC_SH), :],
                send_sem=send_sems.at[2], recv_sem=recv_sems.at[2],
                device_id=nbr, device_id_type=pl.DeviceIdType.MESH),
        ]

    @pl.when(j == 0)
    def _():
        barrier = pltpu.get_barrier_semaphore()
        pl.semaphore_signal(barrier, inc=1, device_id=nbr,
                            device_id_type=pl.DeviceIdType.MESH)
        pl.semaphore_wait(barrier, 1)

        c_loc = jnp.dot(x_ref[...], wdkv_ref[...],
                        preferred_element_type=jnp.float32)
        c_ref[:, pl.ds(off, DC_SH)] = c_loc.astype(jnp.bfloat16)
        wukf_ref[pl.ds(off, DC_SH), :] = wuk_ref[...].astype(jnp.bfloat16)
        wuvf_ref[pl.ds(off, DC_SH), :] = wuv_ref[...].astype(jnp.bfloat16)
        for cp in _copies():
            cp.start()

        qr_ref[...] = (jnp.dot(x_ref[...], wqr_ref[...],
                               preferred_element_type=jnp.float32)
                       * SCALE).astype(jnp.bfloat16)
        kr_ref[...] = jnp.dot(x_ref[...], wkr_ref[...],
                              preferred_element_type=jnp.float32
                              ).astype(jnp.bfloat16)

    q_ref[...] = (jnp.dot(x_ref[...], wq_ref[...],
                          preferred_element_type=jnp.float32)
                  * SCALE).astype(jnp.bfloat16)
    wob_ref[...] = wo_ref[...].astype(jnp.bfloat16)

    @pl.when(j == NT - 1)
    def _():
        for cp in _copies():
            cp.wait()


def _gather_q(x2d, wdkv, wuk, wuv, wq, wqr, wkr, wo):
    return pl.pallas_call(
        _gather_q_body,
        grid=(NT,),
        in_specs=[
            pl.BlockSpec((BS, D), lambda j: (0, 0)),
            pl.BlockSpec((D, DC_SH), lambda j: (0, 0)),
            pl.BlockSpec((DC_SH, D), lambda j: (0, 0)),
            pl.BlockSpec((DC_SH, D), lambda j: (0, 0)),
            pl.BlockSpec((D, TD), lambda j: (0, j)),
            pl.BlockSpec((D, H * DR), lambda j: (0, 0)),
            pl.BlockSpec((D, DR), lambda j: (0, 0)),
            pl.BlockSpec((D, TD), lambda j: (0, j)),
        ],
        out_specs=(
            pl.BlockSpec((BS, DC), lambda j: (0, 0)),
            pl.BlockSpec((DC, D), lambda j: (0, 0)),
            pl.BlockSpec((DC, D), lambda j: (0, 0)),
            pl.BlockSpec((BS, TD), lambda j: (0, j)),
            pl.BlockSpec((BS, H * DR), lambda j: (0, 0)),
            pl.BlockSpec((BS, DR), lambda j: (0, 0)),
            pl.BlockSpec((D, TD), lambda j: (0, j)),
        ),
        out_shape=(
            jax.ShapeDtypeStruct((BS, DC), jnp.bfloat16),
            jax.ShapeDtypeStruct((DC, D), jnp.bfloat16),
            jax.ShapeDtypeStruct((DC, D), jnp.bfloat16),
            jax.ShapeDtypeStruct((BS, D), jnp.bfloat16),
            jax.ShapeDtypeStruct((BS, H * DR), jnp.bfloat16),
            jax.ShapeDtypeStruct((BS, DR), jnp.bfloat16),
            jax.ShapeDtypeStruct((D, D), jnp.bfloat16),
        ),
        scratch_shapes=[
            pltpu.SemaphoreType.DMA((3,)),
            pltpu.SemaphoreType.DMA((3,)),
        ],
        compiler_params=pltpu.CompilerParams(
            collective_id=0, vmem_limit_bytes=_VMEM_LIMIT),
    )(x2d, wdkv, wuk, wuv, wq, wqr, wkr, wo)


def _attn_body(c_ref, wuk_ref, wuv_ref, q_ref, qr_ref, kr_ref, wo_ref,
               out_ref, o_scratch):
    c = c_ref[...]
    kk = jnp.dot(c, wuk_ref[...],
                 preferred_element_type=jnp.float32).astype(jnp.bfloat16)
    vv = jnp.dot(c, wuv_ref[...],
                 preferred_element_type=jnp.float32).astype(jnp.bfloat16)
    kr = kr_ref[...]
    dn = (((1,), (1,)), ((), ()))
    for h in range(H):
        q = q_ref[:, h * DH:(h + 1) * DH]
        k = kk[:, h * DH:(h + 1) * DH]
        qr = qr_ref[:, h * DR:(h + 1) * DR]
        s = lax.dot_general(q, k, dn, preferred_element_type=jnp.float32)
        s = s + lax.dot_general(qr, kr, dn, preferred_element_type=jnp.float32)
        p = jnp.exp(s)
        rs = 1.0 / jnp.sum(p, axis=-1, keepdims=True)
        o = jnp.dot(p.astype(jnp.bfloat16), vv[:, h * DH:(h + 1) * DH],
                    preferred_element_type=jnp.float32)
        o_scratch[:, h * DH:(h + 1) * DH] = (o * rs).astype(jnp.bfloat16)
    out_ref[...] = jnp.dot(o_scratch[...], wo_ref[...],
                           preferred_element_type=jnp.float32)


def _attention(c, wuk_f, wuv_f, q, qr, kr, wo_b):
    return pl.pallas_call(
        _attn_body,
        grid=(B,),
        in_specs=[
            pl.BlockSpec((S, DC), lambda b: (b, 0)),
            pl.BlockSpec((DC, D), lambda b: (0, 0)),
            pl.BlockSpec((DC, D), lambda b: (0, 0)),
            pl.BlockSpec((S, H * DH), lambda b: (b, 0)),
            pl.BlockSpec((S, H * DR), lambda b: (b, 0)),
            pl.BlockSpec((S, DR), lambda b: (b, 0)),
            pl.BlockSpec((D, D), lambda b: (0, 0)),
        ],
        out_specs=pl.BlockSpec((S, D), lambda b: (b, 0)),
        out_shape=jax.ShapeDtypeStruct((BS, D), jnp.float32),
        scratch_shapes=[pltpu.VMEM((S, H * DH), jnp.bfloat16)],
        compiler_params=pltpu.CompilerParams(vmem_limit_bytes=_VMEM_LIMIT),
    )(c, wuk_f, wuv_f, q, qr, kr, wo_b)


def kernel(x, Wdkv, Wuk, Wuv, Wq, Wqr, Wkr, Wo):
    x2d = x.reshape(BS, D)

    c, wuk_f, wuv_f, q, qr, kr, wo_b = _gather_q(
        x2d, Wdkv, Wuk, Wuv, Wq, Wqr, Wkr, Wo)
    out = _attention(c, wuk_f, wuv_f, q, qr, kr, wo_b)
    return out.reshape(B, S, D)
